# baseline (speedup 1.0000x reference)
import functools

import jax
import jax.numpy as jnp
from jax import lax
from jax.experimental import pallas as pl
from jax.experimental.pallas import tpu as pltpu
from jax.experimental.pallas import tpu_sc as plsc


def kernel(rel_pos_emb, length):
    V, D = rel_pos_emb.shape
    L = (V - 1) // 2
    NC, NS = 2, 16
    NW = NC * NS
    ROWT = L * D // 128
    NPHASE = 8
    CPYT = (V * D + 127) // 128 + 1
    rows_per_w = L // NW
    K = 8

    mesh = plsc.VectorSubcoreMesh(core_axis_name="c", subcore_axis_name="s")

    @functools.partial(
        pl.kernel,
        mesh=mesh,
        out_type=jax.ShapeDtypeStruct((L * L * D // 128, 128), jnp.float32),
        scratch_types=[
            pltpu.VMEM_SHARED((NPHASE * CPYT, 128), jnp.float32),
            pltpu.SemaphoreType.DMA,
        ],
    )
    def expand(ext_hbm, out_hbm, table_sh, sem):
        s = lax.axis_index("s")
        wid = s * NC + lax.axis_index("c")
        phase = wid % NPHASE
        q = wid // NPHASE
        i0 = (NPHASE - phase) % NPHASE
        shift = (128 - 16 * phase) % 128

        @pl.when(s == 0)
        def _load():
            pltpu.sync_copy(ext_hbm, table_sh)

        plsc.subcore_barrier()

        def fire(r):
            i = i0 + NPHASE * (q * rows_per_w + r)
            start_t = phase * CPYT + (shift + (L - i) * D) // 128
            pltpu.async_copy(
                table_sh.at[pl.ds(start_t, ROWT), :],
                out_hbm.at[pl.ds(i * ROWT, ROWT), :],
                sem,
            )

        def wait_one():
            pltpu.make_async_copy(
                table_sh.at[pl.ds(0, ROWT), :],
                out_hbm.at[pl.ds(0, ROWT), :],
                sem,
            ).wait()

        def prologue(r, carry):
            fire(r)
            return carry

        def steady(r, carry):
            wait_one()
            fire(r)
            return carry

        def drain(r, carry):
            wait_one()
            return carry

        lax.fori_loop(0, K, prologue, 0)
        lax.fori_loop(K, rows_per_w, steady, 0)
        lax.fori_loop(0, K, drain, 0)

    flat = rel_pos_emb.reshape(V * D)
    copies = [
        jnp.pad(flat, ((128 - 16 * p) % 128, CPYT * 128 - V * D - (128 - 16 * p) % 128))
        for p in range(NPHASE)
    ]
    ext = jnp.concatenate(copies).reshape(NPHASE * CPYT, 128)
    return expand(ext).reshape(L, L, D)


# speedup vs baseline: 2.1264x; 2.1264x over previous
import functools

import jax
import jax.numpy as jnp
from jax.experimental import pallas as pl
from jax.experimental.pallas import tpu as pltpu


def kernel(rel_pos_emb, length):
    V, D = rel_pos_emb.shape
    L = (V - 1) // 2
    CP = 4352
    W = L + 128

    def body(u_ref, out_ref):
        i = pl.program_id(0)
        start = L - i
        base = (start // 128) * 128
        rem = start - base
        win = u_ref[:, pl.ds(pl.multiple_of(base, 128), W)]
        out_ref[...] = pltpu.roll(win, -rem, axis=1)[:, :L]

    expand = pl.pallas_call(
        body,
        grid=(L,),
        in_specs=[pl.BlockSpec((D, CP), lambda i: (0, 0))],
        out_specs=pl.BlockSpec((D, L), lambda i: (i, 0)),
        out_shape=jax.ShapeDtypeStruct((L * D, L), jnp.float32),
    )

    u = jnp.pad(rel_pos_emb, ((0, CP - V), (0, 0))).T
    out2 = expand(u)
    return out2.reshape(L, D, L).transpose(0, 2, 1)


# speedup vs baseline: 2.1266x; 1.0001x over previous
import functools

import jax
import jax.numpy as jnp
from jax.experimental import pallas as pl
from jax.experimental.pallas import tpu as pltpu


def kernel(rel_pos_emb, length):
    V, D = rel_pos_emb.shape
    L = (V - 1) // 2
    CP = 4352
    W = L + 128

    def body(u_ref, out_ref):
        i = pl.program_id(0)
        start = L - i
        base = (start // 128) * 128
        rem = start - base
        win = u_ref[:, pl.ds(pl.multiple_of(base, 128), W)]
        out_ref[...] = pltpu.roll(win, W - rem, axis=1)[:, :L]

    expand = pl.pallas_call(
        body,
        grid=(L,),
        in_specs=[pl.BlockSpec((D, CP), lambda i: (0, 0))],
        out_specs=pl.BlockSpec((D, L), lambda i: (i, 0)),
        out_shape=jax.ShapeDtypeStruct((L * D, L), jnp.float32),
    )

    u = jnp.pad(rel_pos_emb, ((0, CP - V), (0, 0))).T
    out2 = expand(u)
    return out2.reshape(L, D, L).transpose(0, 2, 1)


# speedup vs baseline: 9.5266x; 4.4798x over previous
import functools

import jax
import jax.numpy as jnp
from jax.experimental import pallas as pl
from jax.experimental.pallas import tpu as pltpu


def kernel(rel_pos_emb, length):
    V, D = rel_pos_emb.shape
    L = (V - 1) // 2
    CP = 4352
    W = L + 128

    R = 8

    def body(u_ref, out_ref):
        g = pl.program_id(0)
        for r in range(R):
            i = g * R + r
            start = L - i
            base = (start // 128) * 128
            rem = start - base
            win = u_ref[:, pl.ds(pl.multiple_of(base, 128), W)]
            out_ref[pl.ds(r * D, D), :] = pltpu.roll(win, W - rem, axis=1)[:, :L]

    expand = pl.pallas_call(
        body,
        grid=(L // R,),
        in_specs=[pl.BlockSpec((D, CP), lambda i: (0, 0))],
        out_specs=pl.BlockSpec((R * D, L), lambda i: (i, 0)),
        out_shape=jax.ShapeDtypeStruct((L * D, L), jnp.float32),
    )

    u = jnp.pad(rel_pos_emb, ((0, CP - V), (0, 0))).T
    out2 = expand(u)
    return out2.reshape(L, D, L).transpose(0, 2, 1)


# speedup vs baseline: 12.9222x; 1.3564x over previous
import functools

import jax
import jax.numpy as jnp
from jax.experimental import pallas as pl
from jax.experimental.pallas import tpu as pltpu


def kernel(rel_pos_emb, length):
    V, D = rel_pos_emb.shape
    L = (V - 1) // 2
    CP = 4352
    W = L + 128

    R = 16

    def body(u_ref, out_ref):
        g = pl.program_id(0)
        for r in range(R):
            i = g * R + r
            start = L - i
            base = (start // 128) * 128
            rem = start - base
            win = u_ref[:, pl.ds(pl.multiple_of(base, 128), W)]
            out_ref[pl.ds(r * D, D), :] = pltpu.roll(win, W - rem, axis=1)[:, :L]

    expand = pl.pallas_call(
        body,
        grid=(L // R,),
        in_specs=[pl.BlockSpec((D, CP), lambda i: (0, 0))],
        out_specs=pl.BlockSpec((R * D, L), lambda i: (i, 0)),
        out_shape=jax.ShapeDtypeStruct((L * D, L), jnp.float32),
    )

    u = jnp.pad(rel_pos_emb, ((0, CP - V), (0, 0))).T
    out2 = expand(u)
    return out2.reshape(L, D, L).transpose(0, 2, 1)


# speedup vs baseline: 15.8080x; 1.2233x over previous
import functools

import jax
import jax.numpy as jnp
from jax.experimental import pallas as pl
from jax.experimental.pallas import tpu as pltpu


def kernel(rel_pos_emb, length):
    V, D = rel_pos_emb.shape
    L = (V - 1) // 2
    CP = 4352
    W = L + 128

    R = 32

    def body(u_ref, out_ref):
        g = pl.program_id(0)
        for r in range(R):
            i = g * R + r
            start = L - i
            base = (start // 128) * 128
            rem = start - base
            win = u_ref[:, pl.ds(pl.multiple_of(base, 128), W)]
            out_ref[pl.ds(r * D, D), :] = pltpu.roll(win, W - rem, axis=1)[:, :L]

    expand = pl.pallas_call(
        body,
        grid=(L // R,),
        in_specs=[pl.BlockSpec((D, CP), lambda i: (0, 0))],
        out_specs=pl.BlockSpec((R * D, L), lambda i: (i, 0)),
        out_shape=jax.ShapeDtypeStruct((L * D, L), jnp.float32),
    )

    u = jnp.pad(rel_pos_emb, ((0, CP - V), (0, 0))).T
    out2 = expand(u)
    return out2.reshape(L, D, L).transpose(0, 2, 1)


# speedup vs baseline: 17.7500x; 1.1228x over previous
import functools

import jax
import jax.numpy as jnp
from jax.experimental import pallas as pl
from jax.experimental.pallas import tpu as pltpu


def kernel(rel_pos_emb, length):
    V, D = rel_pos_emb.shape
    L = (V - 1) // 2
    CP = 4352
    W = L + 128

    R = 64

    def body(u_ref, out_ref):
        g = pl.program_id(0)
        for r in range(R):
            i = g * R + r
            start = L - i
            base = (start // 128) * 128
            rem = start - base
            win = u_ref[:, pl.ds(pl.multiple_of(base, 128), W)]
            out_ref[pl.ds(r * D, D), :] = pltpu.roll(win, W - rem, axis=1)[:, :L]

    expand = pl.pallas_call(
        body,
        grid=(L // R,),
        in_specs=[pl.BlockSpec((D, CP), lambda i: (0, 0))],
        out_specs=pl.BlockSpec((R * D, L), lambda i: (i, 0)),
        out_shape=jax.ShapeDtypeStruct((L * D, L), jnp.float32),
    )

    u = jnp.pad(rel_pos_emb, ((0, CP - V), (0, 0))).T
    out2 = expand(u)
    return out2.reshape(L, D, L).transpose(0, 2, 1)


# speedup vs baseline: 18.2811x; 1.0299x over previous
import functools

import jax
import jax.numpy as jnp
from jax.experimental import pallas as pl
from jax.experimental.pallas import tpu as pltpu


def kernel(rel_pos_emb, length):
    V, D = rel_pos_emb.shape
    L = (V - 1) // 2
    CP = 4352
    W = L + 128

    R = 128

    def body(u_ref, out_ref):
        g = pl.program_id(0)
        for r in range(R):
            i = g * R + r
            start = L - i
            base = (start // 128) * 128
            rem = start - base
            win = u_ref[:, pl.ds(pl.multiple_of(base, 128), W)]
            out_ref[pl.ds(r * D, D), :] = pltpu.roll(win, W - rem, axis=1)[:, :L]

    expand = pl.pallas_call(
        body,
        grid=(L // R,),
        in_specs=[pl.BlockSpec((D, CP), lambda i: (0, 0))],
        out_specs=pl.BlockSpec((R * D, L), lambda i: (i, 0)),
        out_shape=jax.ShapeDtypeStruct((L * D, L), jnp.float32),
    )

    u = jnp.pad(rel_pos_emb, ((0, CP - V), (0, 0))).T
    out2 = expand(u)
    return out2.reshape(L, D, L).transpose(0, 2, 1)


# speedup vs baseline: 18.5228x; 1.0132x over previous
import functools

import jax
import jax.numpy as jnp
from jax.experimental import pallas as pl
from jax.experimental.pallas import tpu as pltpu


def kernel(rel_pos_emb, length):
    V, D = rel_pos_emb.shape
    L = (V - 1) // 2
    CP = 4352
    W = L + 128

    R = 128

    def body(u_ref, out_ref):
        g = pl.program_id(0)
        lane = jax.lax.broadcasted_iota(jnp.int32, (D, 128), 1)
        for r in range(R):
            i = g * R + r
            start = L - i
            base = (start // 128) * 128
            rem = start - base
            win = u_ref[:, pl.ds(pl.multiple_of(base, 128), W)]
            sh = 128 - rem
            mask = lane < sh
            rots = [
                pltpu.roll(win[:, c * 128:(c + 1) * 128], sh, axis=1)
                for c in range(W // 128)
            ]
            row = jnp.concatenate(
                [jnp.where(mask, rots[c], rots[c + 1]) for c in range(L // 128)],
                axis=1,
            )
            out_ref[pl.ds(r * D, D), :] = row

    expand = pl.pallas_call(
        body,
        grid=(L // R,),
        in_specs=[pl.BlockSpec((D, CP), lambda i: (0, 0))],
        out_specs=pl.BlockSpec((R * D, L), lambda i: (i, 0)),
        out_shape=jax.ShapeDtypeStruct((L * D, L), jnp.float32),
    )

    u = jnp.pad(rel_pos_emb, ((0, CP - V), (0, 0))).T
    out2 = expand(u)
    return out2.reshape(L, D, L).transpose(0, 2, 1)
